# baseline (speedup 1.0000x reference)
import functools

import jax
import jax.numpy as jnp
from jax.experimental import pallas as pl
from jax.experimental.pallas import tpu as pltpu

_EPS = 1e-5
_VMEM_LIMIT = 64 * 1024 * 1024
_HIGH = jax.lax.Precision.HIGHEST


def _dw_gram_kernel(x_ref, w_ref, b_ref, z_ref, g_ref, s_ref, pad_ref,
                    *, ho, wo, ksize):
    xin = x_ref[0]
    c = xin.shape[-1]
    for wp in range(2):
        for hp in range(2):
            r0, c0 = 1 - hp, 1 - wp
            pad_ref[wp, hp, r0:r0 + ho, c0:c0 + wo, :] = xin[1 - wp, 1 - hp]
            pad_ref[wp, hp, hp * ho:hp * ho + 1, :, :] = jnp.zeros(
                (1, wo + 1, c), jnp.float32)
            pad_ref[wp, hp, :, wp * wo:wp * wo + 1, :] = jnp.zeros(
                (wo + 1, 1, c), jnp.float32)
    acc = jnp.zeros((ho, wo, c), jnp.float32) + b_ref[...]
    for kh in range(ksize):
        ph, a0 = kh % 2, kh // 2
        for kw in range(ksize):
            pw, b0 = kw % 2, kw // 2
            xs = pad_ref[pw, ph, a0:a0 + ho, b0:b0 + wo, :]
            acc = acc + w_ref[kh * ksize + kw] * xs
    z_ref[0] = acc
    zv = acc.reshape(ho * wo, acc.shape[-1])
    g_ref[0] = jax.lax.dot_general(zv, zv, (((0,), (0,)), ((), ())),
                                   preferred_element_type=jnp.float32,
                                   precision=_HIGH)
    s_ref[0] = jnp.sum(zv, axis=0, keepdims=True)


def _finalize_kernel(g_ref, s_ref, w2t_ref, gamma_ref, beta_ref,
                     scale_ref, shift_ref, *, count):
    g = jnp.sum(g_ref[...], axis=0)
    zs = jnp.sum(s_ref[...], axis=0)
    w2t = w2t_ref[...]
    syh = jax.lax.dot_general(zs, w2t, (((1,), (0,)), ((), ())),
                              preferred_element_type=jnp.float32,
                              precision=_HIGH)
    a = jax.lax.dot_general(g, w2t, (((1,), (0,)), ((), ())),
                            preferred_element_type=jnp.float32,
                            precision=_HIGH)
    ssq = jnp.sum(a * w2t, axis=0, keepdims=True)
    inv_cnt = jnp.float32(1.0 / count)
    mean_hat = syh * inv_cnt
    var = jnp.maximum(ssq * inv_cnt - mean_hat * mean_hat, 0.0)
    scale = gamma_ref[...] * jax.lax.rsqrt(var + _EPS)
    shift = beta_ref[...] - mean_hat * scale
    scale_ref[...] = scale.reshape(scale_ref.shape)
    shift_ref[...] = shift.reshape(shift_ref.shape)


def _apply_kernel(w2_ref, scale_ref, shift_ref, z_ref, o_ref, *, hw):
    zv = z_ref[0].reshape(hw, z_ref.shape[-1])
    y = jax.lax.dot_general(w2_ref[...], zv, (((1,), (1,)), ((), ())),
                            preferred_element_type=jnp.float32,
                            precision=_HIGH)
    y = y * scale_ref[...] + shift_ref[...]
    y = jnp.maximum(y, 0.0).astype(o_ref.dtype)
    o_ref[0] = y.reshape(o_ref.shape[1:])


def kernel(x, dw_w, dw_b, pw_w, pw_b, gamma, beta):
    del pw_b
    n, cin, h, w = x.shape
    cout = pw_w.shape[0]
    ksize = dw_w.shape[-1]
    pad = 1
    ho = (h + 2 * pad - ksize) // 2 + 1
    wo = (w + 2 * pad - ksize) // 2 + 1
    hw = ho * wo

    xq = x.reshape(n, cin, ho, 2, wo, 2).transpose(0, 5, 3, 2, 4, 1)
    wt = dw_w.reshape(cin, ksize * ksize).T.reshape(ksize * ksize, 1, cin)
    wt = wt.astype(jnp.float32)
    bt = dw_b.reshape(1, 1, cin).astype(jnp.float32)
    w2 = pw_w.reshape(cout, cin).astype(jnp.float32)
    w2t = w2.T

    dw_kern = functools.partial(_dw_gram_kernel, ho=ho, wo=wo, ksize=ksize)
    z, gmat, ssum = pl.pallas_call(
        dw_kern,
        out_shape=[
            jax.ShapeDtypeStruct((n, ho, wo, cin), jnp.float32),
            jax.ShapeDtypeStruct((n, cin, cin), jnp.float32),
            jax.ShapeDtypeStruct((n, 1, cin), jnp.float32),
        ],
        grid=(n,),
        in_specs=[
            pl.BlockSpec((1, 2, 2, ho, wo, cin), lambda nn: (nn, 0, 0, 0, 0, 0)),
            pl.BlockSpec((ksize * ksize, 1, cin), lambda nn: (0, 0, 0)),
            pl.BlockSpec((1, 1, cin), lambda nn: (0, 0, 0)),
        ],
        scratch_shapes=[
            pltpu.VMEM((2, 2, ho + 1, wo + 1, cin), jnp.float32),
        ],
        out_specs=[
            pl.BlockSpec((1, ho, wo, cin), lambda nn: (nn, 0, 0, 0)),
            pl.BlockSpec((1, cin, cin), lambda nn: (nn, 0, 0)),
            pl.BlockSpec((1, 1, cin), lambda nn: (nn, 0, 0)),
        ],
        compiler_params=pltpu.CompilerParams(
            dimension_semantics=("parallel",),
            vmem_limit_bytes=_VMEM_LIMIT,
        ),
    )(xq, wt, bt)

    fin_kern = functools.partial(_finalize_kernel, count=n * hw)
    scale, shift = pl.pallas_call(
        fin_kern,
        out_shape=[jax.ShapeDtypeStruct((cout, 1), jnp.float32)] * 2,
        grid=(1,),
        in_specs=[
            pl.BlockSpec((n, cin, cin), lambda i: (0, 0, 0)),
            pl.BlockSpec((n, 1, cin), lambda i: (0, 0, 0)),
            pl.BlockSpec((cin, cout), lambda i: (0, 0)),
            pl.BlockSpec((1, cout), lambda i: (0, 0)),
            pl.BlockSpec((1, cout), lambda i: (0, 0)),
        ],
        out_specs=[pl.BlockSpec((cout, 1), lambda i: (0, 0))] * 2,
        compiler_params=pltpu.CompilerParams(
            dimension_semantics=("arbitrary",),
            vmem_limit_bytes=_VMEM_LIMIT,
        ),
    )(gmat, ssum, w2t, gamma.reshape(1, cout).astype(jnp.float32),
      beta.reshape(1, cout).astype(jnp.float32))

    ap_kern = functools.partial(_apply_kernel, hw=hw)
    out4 = pl.pallas_call(
        ap_kern,
        out_shape=jax.ShapeDtypeStruct((n, cout, ho, wo), x.dtype),
        grid=(n,),
        in_specs=[
            pl.BlockSpec((cout, cin), lambda nn: (0, 0)),
            pl.BlockSpec((cout, 1), lambda nn: (0, 0)),
            pl.BlockSpec((cout, 1), lambda nn: (0, 0)),
            pl.BlockSpec((1, ho, wo, cin), lambda nn: (nn, 0, 0, 0)),
        ],
        out_specs=pl.BlockSpec((1, cout, ho, wo), lambda nn: (nn, 0, 0, 0)),
        compiler_params=pltpu.CompilerParams(
            dimension_semantics=("parallel",),
            vmem_limit_bytes=_VMEM_LIMIT,
        ),
    )(w2, scale, shift, z)
    return out4


# speedup vs baseline: 1.2217x; 1.2217x over previous
import functools

import jax
import jax.numpy as jnp
from jax.experimental import pallas as pl
from jax.experimental.pallas import tpu as pltpu

_EPS = 1e-5
_VMEM_LIMIT = 64 * 1024 * 1024
_HIGH = jax.lax.Precision.HIGHEST


def _dw_gram_kernel(x_ref, w_ref, b_ref, z_ref, g_ref, s_ref, pad_ref,
                    *, ho, wo, ksize):
    xin = x_ref[0]
    c = xin.shape[-1]
    for wp in range(2):
        for hp in range(2):
            r0, c0 = 1 - hp, 1 - wp
            pad_ref[wp, hp, r0:r0 + ho, c0:c0 + wo, :] = xin[1 - wp, 1 - hp]
            pad_ref[wp, hp, hp * ho:hp * ho + 1, :, :] = jnp.zeros(
                (1, wo + 1, c), pad_ref.dtype)
            pad_ref[wp, hp, :, wp * wo:wp * wo + 1, :] = jnp.zeros(
                (wo + 1, 1, c), pad_ref.dtype)
    acc = jnp.zeros((ho, wo, c), jnp.float32) + b_ref[...]
    for kh in range(ksize):
        ph, a0 = kh % 2, kh // 2
        for kw in range(ksize):
            pw, b0 = kw % 2, kw // 2
            xs = pad_ref[pw, ph, a0:a0 + ho, b0:b0 + wo, :]
            acc = acc + w_ref[kh * ksize + kw] * xs.astype(jnp.float32)
    z_ref[0] = acc.astype(z_ref.dtype)
    zv = acc.reshape(ho * wo, acc.shape[-1])
    g_ref[0] = jax.lax.dot_general(zv, zv, (((0,), (0,)), ((), ())),
                                   preferred_element_type=jnp.float32,
                                   precision=_HIGH)
    s_ref[0] = jnp.sum(zv, axis=0, keepdims=True)


def _finalize_kernel(g_ref, s_ref, w2t_ref, gamma_ref, beta_ref,
                     scale_ref, shift_ref, *, count):
    g = jnp.sum(g_ref[...], axis=0)
    zs = jnp.sum(s_ref[...], axis=0)
    w2t = w2t_ref[...]
    syh = jax.lax.dot_general(zs, w2t, (((1,), (0,)), ((), ())),
                              preferred_element_type=jnp.float32,
                              precision=_HIGH)
    a = jax.lax.dot_general(g, w2t, (((1,), (0,)), ((), ())),
                            preferred_element_type=jnp.float32,
                            precision=_HIGH)
    ssq = jnp.sum(a * w2t, axis=0, keepdims=True)
    inv_cnt = jnp.float32(1.0 / count)
    mean_hat = syh * inv_cnt
    var = jnp.maximum(ssq * inv_cnt - mean_hat * mean_hat, 0.0)
    scale = gamma_ref[...] * jax.lax.rsqrt(var + _EPS)
    shift = beta_ref[...] - mean_hat * scale
    scale_ref[...] = scale.reshape(scale_ref.shape)
    shift_ref[...] = shift.reshape(shift_ref.shape)


def _apply_kernel(w2_ref, scale_ref, shift_ref, z_ref, o_ref, *, hw):
    zv = z_ref[0].astype(jnp.float32).reshape(hw, z_ref.shape[-1])
    y = jax.lax.dot_general(w2_ref[...], zv, (((1,), (1,)), ((), ())),
                            preferred_element_type=jnp.float32,
                            precision=_HIGH)
    y = y * scale_ref[...] + shift_ref[...]
    o_ref[0] = jnp.maximum(y, 0.0).astype(o_ref.dtype)


def kernel(x, dw_w, dw_b, pw_w, pw_b, gamma, beta):
    del pw_b
    n, cin, h, w = x.shape
    cout = pw_w.shape[0]
    ksize = dw_w.shape[-1]
    pad = 1
    ho = (h + 2 * pad - ksize) // 2 + 1
    wo = (w + 2 * pad - ksize) // 2 + 1
    hw = ho * wo

    xq = x.reshape(n, cin, ho, 2, wo, 2).transpose(0, 5, 3, 2, 4, 1)
    xq = xq.astype(jnp.bfloat16)
    wt = dw_w.reshape(cin, ksize * ksize).T.reshape(ksize * ksize, 1, cin)
    wt = wt.astype(jnp.float32)
    bt = dw_b.reshape(1, 1, cin).astype(jnp.float32)
    w2 = pw_w.reshape(cout, cin).astype(jnp.float32)
    w2t = w2.T

    dw_kern = functools.partial(_dw_gram_kernel, ho=ho, wo=wo, ksize=ksize)
    z, gmat, ssum = pl.pallas_call(
        dw_kern,
        out_shape=[
            jax.ShapeDtypeStruct((n, ho, wo, cin), jnp.bfloat16),
            jax.ShapeDtypeStruct((n, cin, cin), jnp.float32),
            jax.ShapeDtypeStruct((n, 1, cin), jnp.float32),
        ],
        grid=(n,),
        in_specs=[
            pl.BlockSpec((1, 2, 2, ho, wo, cin), lambda nn: (nn, 0, 0, 0, 0, 0)),
            pl.BlockSpec((ksize * ksize, 1, cin), lambda nn: (0, 0, 0)),
            pl.BlockSpec((1, 1, cin), lambda nn: (0, 0, 0)),
        ],
        scratch_shapes=[
            pltpu.VMEM((2, 2, ho + 1, wo + 1, cin), jnp.bfloat16),
        ],
        out_specs=[
            pl.BlockSpec((1, ho, wo, cin), lambda nn: (nn, 0, 0, 0)),
            pl.BlockSpec((1, cin, cin), lambda nn: (nn, 0, 0)),
            pl.BlockSpec((1, 1, cin), lambda nn: (nn, 0, 0)),
        ],
        compiler_params=pltpu.CompilerParams(
            dimension_semantics=("parallel",),
            vmem_limit_bytes=_VMEM_LIMIT,
        ),
    )(xq, wt, bt)

    fin_kern = functools.partial(_finalize_kernel, count=n * hw)
    scale, shift = pl.pallas_call(
        fin_kern,
        out_shape=[jax.ShapeDtypeStruct((cout, 1), jnp.float32)] * 2,
        grid=(1,),
        in_specs=[
            pl.BlockSpec((n, cin, cin), lambda i: (0, 0, 0)),
            pl.BlockSpec((n, 1, cin), lambda i: (0, 0, 0)),
            pl.BlockSpec((cin, cout), lambda i: (0, 0)),
            pl.BlockSpec((1, cout), lambda i: (0, 0)),
            pl.BlockSpec((1, cout), lambda i: (0, 0)),
        ],
        out_specs=[pl.BlockSpec((cout, 1), lambda i: (0, 0))] * 2,
        compiler_params=pltpu.CompilerParams(
            dimension_semantics=("arbitrary",),
            vmem_limit_bytes=_VMEM_LIMIT,
        ),
    )(gmat, ssum, w2t, gamma.reshape(1, cout).astype(jnp.float32),
      beta.reshape(1, cout).astype(jnp.float32))

    ap_kern = functools.partial(_apply_kernel, hw=hw)
    out3 = pl.pallas_call(
        ap_kern,
        out_shape=jax.ShapeDtypeStruct((n, cout, hw), x.dtype),
        grid=(n,),
        in_specs=[
            pl.BlockSpec((cout, cin), lambda nn: (0, 0)),
            pl.BlockSpec((cout, 1), lambda nn: (0, 0)),
            pl.BlockSpec((cout, 1), lambda nn: (0, 0)),
            pl.BlockSpec((1, ho, wo, cin), lambda nn: (nn, 0, 0, 0)),
        ],
        out_specs=pl.BlockSpec((1, cout, hw), lambda nn: (nn, 0, 0)),
        compiler_params=pltpu.CompilerParams(
            dimension_semantics=("parallel",),
            vmem_limit_bytes=_VMEM_LIMIT,
        ),
    )(w2, scale, shift, z)
    return out3.reshape(n, cout, ho, wo)


# speedup vs baseline: 1.3465x; 1.1021x over previous
import functools

import jax
import jax.numpy as jnp
from jax.experimental import pallas as pl
from jax.experimental.pallas import tpu as pltpu

_EPS = 1e-5
_VMEM_LIMIT = 64 * 1024 * 1024
_HIGH = jax.lax.Precision.HIGHEST


def _dw_gram_kernel(x_ref, w_ref, b_ref, z_ref, g_ref, s_ref, pad_ref,
                    *, ho, wo, ksize):
    xin = x_ref[0]
    c = xin.shape[-1]
    for wp in range(2):
        for hp in range(2):
            r0, c0 = 1 - hp, 1 - wp
            pad_ref[wp, hp, r0:r0 + ho, c0:c0 + wo, :] = xin[1 - wp, 1 - hp]
            pad_ref[wp, hp, hp * ho:hp * ho + 1, :, :] = jnp.zeros(
                (1, wo + 1, c), jnp.float32)
            pad_ref[wp, hp, :, wp * wo:wp * wo + 1, :] = jnp.zeros(
                (wo + 1, 1, c), jnp.float32)
    acc = jnp.zeros((ho, wo, c), jnp.float32) + b_ref[...]
    for kh in range(ksize):
        ph, a0 = kh % 2, kh // 2
        for kw in range(ksize):
            pw, b0 = kw % 2, kw // 2
            xs = pad_ref[pw, ph, a0:a0 + ho, b0:b0 + wo, :]
            acc = acc + w_ref[kh * ksize + kw] * xs
    z_ref[0] = acc
    zv = acc.reshape(ho * wo, acc.shape[-1])
    g_ref[0] = jax.lax.dot_general(zv, zv, (((0,), (0,)), ((), ())),
                                   preferred_element_type=jnp.float32,
                                   precision=_HIGH)
    s_ref[0] = jnp.sum(zv, axis=0, keepdims=True)


def _finalize_kernel(g_ref, s_ref, w2t_ref, gamma_ref, beta_ref,
                     scale_ref, shift_ref, *, count):
    g = jnp.sum(g_ref[...], axis=0)
    zs = jnp.sum(s_ref[...], axis=0)
    w2t = w2t_ref[...]
    syh = jax.lax.dot_general(zs, w2t, (((1,), (0,)), ((), ())),
                              preferred_element_type=jnp.float32,
                              precision=_HIGH)
    a = jax.lax.dot_general(g, w2t, (((1,), (0,)), ((), ())),
                            preferred_element_type=jnp.float32,
                            precision=_HIGH)
    ssq = jnp.sum(a * w2t, axis=0, keepdims=True)
    inv_cnt = jnp.float32(1.0 / count)
    mean_hat = syh * inv_cnt
    var = jnp.maximum(ssq * inv_cnt - mean_hat * mean_hat, 0.0)
    scale = gamma_ref[...] * jax.lax.rsqrt(var + _EPS)
    shift = beta_ref[...] - mean_hat * scale
    scale_ref[...] = scale.reshape(scale_ref.shape)
    shift_ref[...] = shift.reshape(shift_ref.shape)


def _apply_kernel(w2_ref, scale_ref, shift_ref, z_ref, o_ref, *, hw):
    zv = z_ref[0].reshape(hw, z_ref.shape[-1])
    y = jax.lax.dot_general(w2_ref[...], zv, (((1,), (1,)), ((), ())),
                            preferred_element_type=jnp.float32,
                            precision=_HIGH)
    y = y * scale_ref[...] + shift_ref[...]
    o_ref[0] = jnp.maximum(y, 0.0).astype(o_ref.dtype)


def kernel(x, dw_w, dw_b, pw_w, pw_b, gamma, beta):
    del pw_b
    n, cin, h, w = x.shape
    cout = pw_w.shape[0]
    ksize = dw_w.shape[-1]
    pad = 1
    ho = (h + 2 * pad - ksize) // 2 + 1
    wo = (w + 2 * pad - ksize) // 2 + 1
    hw = ho * wo

    xq = x.reshape(n, cin, ho, 2, wo, 2).transpose(0, 5, 3, 2, 4, 1)
    wt = dw_w.reshape(cin, ksize * ksize).T.reshape(ksize * ksize, 1, cin)
    wt = wt.astype(jnp.float32)
    bt = dw_b.reshape(1, 1, cin).astype(jnp.float32)
    w2 = pw_w.reshape(cout, cin).astype(jnp.float32)
    w2t = w2.T

    dw_kern = functools.partial(_dw_gram_kernel, ho=ho, wo=wo, ksize=ksize)
    z, gmat, ssum = pl.pallas_call(
        dw_kern,
        out_shape=[
            jax.ShapeDtypeStruct((n, ho, wo, cin), jnp.float32),
            jax.ShapeDtypeStruct((n, cin, cin), jnp.float32),
            jax.ShapeDtypeStruct((n, 1, cin), jnp.float32),
        ],
        grid=(n,),
        in_specs=[
            pl.BlockSpec((1, 2, 2, ho, wo, cin), lambda nn: (nn, 0, 0, 0, 0, 0)),
            pl.BlockSpec((ksize * ksize, 1, cin), lambda nn: (0, 0, 0)),
            pl.BlockSpec((1, 1, cin), lambda nn: (0, 0, 0)),
        ],
        scratch_shapes=[
            pltpu.VMEM((2, 2, ho + 1, wo + 1, cin), jnp.float32),
        ],
        out_specs=[
            pl.BlockSpec((1, ho, wo, cin), lambda nn: (nn, 0, 0, 0)),
            pl.BlockSpec((1, cin, cin), lambda nn: (nn, 0, 0)),
            pl.BlockSpec((1, 1, cin), lambda nn: (nn, 0, 0)),
        ],
        compiler_params=pltpu.CompilerParams(
            dimension_semantics=("parallel",),
            vmem_limit_bytes=_VMEM_LIMIT,
        ),
    )(xq, wt, bt)

    fin_kern = functools.partial(_finalize_kernel, count=n * hw)
    scale, shift = pl.pallas_call(
        fin_kern,
        out_shape=[jax.ShapeDtypeStruct((cout, 1), jnp.float32)] * 2,
        grid=(1,),
        in_specs=[
            pl.BlockSpec((n, cin, cin), lambda i: (0, 0, 0)),
            pl.BlockSpec((n, 1, cin), lambda i: (0, 0, 0)),
            pl.BlockSpec((cin, cout), lambda i: (0, 0)),
            pl.BlockSpec((1, cout), lambda i: (0, 0)),
            pl.BlockSpec((1, cout), lambda i: (0, 0)),
        ],
        out_specs=[pl.BlockSpec((cout, 1), lambda i: (0, 0))] * 2,
        compiler_params=pltpu.CompilerParams(
            dimension_semantics=("arbitrary",),
            vmem_limit_bytes=_VMEM_LIMIT,
        ),
    )(gmat, ssum, w2t, gamma.reshape(1, cout).astype(jnp.float32),
      beta.reshape(1, cout).astype(jnp.float32))

    ap_kern = functools.partial(_apply_kernel, hw=hw)
    out3 = pl.pallas_call(
        ap_kern,
        out_shape=jax.ShapeDtypeStruct((n, cout, hw), x.dtype),
        grid=(n,),
        in_specs=[
            pl.BlockSpec((cout, cin), lambda nn: (0, 0)),
            pl.BlockSpec((cout, 1), lambda nn: (0, 0)),
            pl.BlockSpec((cout, 1), lambda nn: (0, 0)),
            pl.BlockSpec((1, ho, wo, cin), lambda nn: (nn, 0, 0, 0)),
        ],
        out_specs=pl.BlockSpec((1, cout, hw), lambda nn: (nn, 0, 0)),
        compiler_params=pltpu.CompilerParams(
            dimension_semantics=("parallel",),
            vmem_limit_bytes=_VMEM_LIMIT,
        ),
    )(w2, scale, shift, z)
    return out3.reshape(n, cout, ho, wo)


# speedup vs baseline: 1.4726x; 1.0937x over previous
import functools

import jax
import jax.numpy as jnp
from jax.experimental import pallas as pl
from jax.experimental.pallas import tpu as pltpu

_EPS = 1e-5
_VMEM_LIMIT = 64 * 1024 * 1024
_HIGH = jax.lax.Precision.HIGHEST


def _dw_gram_kernel(x_ref, w_ref, b_ref, z_ref, g_ref, s_ref, pad_ref,
                    *, ho, wo, ksize):
    c = x_ref.shape[-1]
    for i in range(x_ref.shape[0]):
        xin = x_ref[i]
        for wp in range(2):
            for hp in range(2):
                r0, c0 = 1 - hp, 1 - wp
                pad_ref[wp, hp, r0:r0 + ho, c0:c0 + wo, :] = xin[1 - wp, 1 - hp]
                pad_ref[wp, hp, hp * ho:hp * ho + 1, :, :] = jnp.zeros(
                    (1, wo + 1, c), jnp.float32)
                pad_ref[wp, hp, :, wp * wo:wp * wo + 1, :] = jnp.zeros(
                    (wo + 1, 1, c), jnp.float32)
        acc = jnp.zeros((ho, wo, c), jnp.float32) + b_ref[...]
        for kh in range(ksize):
            ph, a0 = kh % 2, kh // 2
            for kw in range(ksize):
                pw, b0 = kw % 2, kw // 2
                xs = pad_ref[pw, ph, a0:a0 + ho, b0:b0 + wo, :]
                acc = acc + w_ref[kh * ksize + kw] * xs
        z_ref[i] = acc
        zv = acc.reshape(ho * wo, acc.shape[-1])
        g_ref[i] = jax.lax.dot_general(zv, zv, (((0,), (0,)), ((), ())),
                                       preferred_element_type=jnp.float32,
                                       precision=_HIGH)
        s_ref[i] = jnp.sum(zv, axis=0, keepdims=True)


def _finalize_kernel(g_ref, s_ref, w2t_ref, gamma_ref, beta_ref,
                     scale_ref, shift_ref, *, count):
    g = jnp.sum(g_ref[...], axis=0)
    zs = jnp.sum(s_ref[...], axis=0)
    w2t = w2t_ref[...]
    syh = jax.lax.dot_general(zs, w2t, (((1,), (0,)), ((), ())),
                              preferred_element_type=jnp.float32,
                              precision=_HIGH)
    a = jax.lax.dot_general(g, w2t, (((1,), (0,)), ((), ())),
                            preferred_element_type=jnp.float32,
                            precision=_HIGH)
    ssq = jnp.sum(a * w2t, axis=0, keepdims=True)
    inv_cnt = jnp.float32(1.0 / count)
    mean_hat = syh * inv_cnt
    var = jnp.maximum(ssq * inv_cnt - mean_hat * mean_hat, 0.0)
    scale = gamma_ref[...] * jax.lax.rsqrt(var + _EPS)
    shift = beta_ref[...] - mean_hat * scale
    scale_ref[...] = scale.reshape(scale_ref.shape)
    shift_ref[...] = shift.reshape(shift_ref.shape)


def _apply_kernel(w2_ref, scale_ref, shift_ref, z_ref, o_ref, *, hw):
    for i in range(z_ref.shape[0]):
        zv = z_ref[i].reshape(hw, z_ref.shape[-1])
        y = jax.lax.dot_general(w2_ref[...], zv, (((1,), (1,)), ((), ())),
                                preferred_element_type=jnp.float32,
                                precision=_HIGH)
        y = y * scale_ref[...] + shift_ref[...]
        o_ref[i] = jnp.maximum(y, 0.0).astype(o_ref.dtype)


def kernel(x, dw_w, dw_b, pw_w, pw_b, gamma, beta):
    del pw_b
    n, cin, h, w = x.shape
    cout = pw_w.shape[0]
    ksize = dw_w.shape[-1]
    pad = 1
    ho = (h + 2 * pad - ksize) // 2 + 1
    wo = (w + 2 * pad - ksize) // 2 + 1
    hw = ho * wo
    bn = 2 if n % 2 == 0 else 1

    xq = x.reshape(n, cin, ho, 2, wo, 2).transpose(0, 5, 3, 2, 4, 1)
    wt = dw_w.reshape(cin, ksize * ksize).T.reshape(ksize * ksize, 1, cin)
    wt = wt.astype(jnp.float32)
    bt = dw_b.reshape(1, 1, cin).astype(jnp.float32)
    w2 = pw_w.reshape(cout, cin).astype(jnp.float32)
    w2t = w2.T

    dw_kern = functools.partial(_dw_gram_kernel, ho=ho, wo=wo, ksize=ksize)
    z, gmat, ssum = pl.pallas_call(
        dw_kern,
        out_shape=[
            jax.ShapeDtypeStruct((n, ho, wo, cin), jnp.float32),
            jax.ShapeDtypeStruct((n, cin, cin), jnp.float32),
            jax.ShapeDtypeStruct((n, 1, cin), jnp.float32),
        ],
        grid=(n // bn,),
        in_specs=[
            pl.BlockSpec((bn, 2, 2, ho, wo, cin),
                         lambda nn: (nn, 0, 0, 0, 0, 0)),
            pl.BlockSpec((ksize * ksize, 1, cin), lambda nn: (0, 0, 0)),
            pl.BlockSpec((1, 1, cin), lambda nn: (0, 0, 0)),
        ],
        scratch_shapes=[
            pltpu.VMEM((2, 2, ho + 1, wo + 1, cin), jnp.float32),
        ],
        out_specs=[
            pl.BlockSpec((bn, ho, wo, cin), lambda nn: (nn, 0, 0, 0)),
            pl.BlockSpec((bn, cin, cin), lambda nn: (nn, 0, 0)),
            pl.BlockSpec((bn, 1, cin), lambda nn: (nn, 0, 0)),
        ],
        compiler_params=pltpu.CompilerParams(
            dimension_semantics=("parallel",),
            vmem_limit_bytes=_VMEM_LIMIT,
        ),
    )(xq, wt, bt)

    fin_kern = functools.partial(_finalize_kernel, count=n * hw)
    scale, shift = pl.pallas_call(
        fin_kern,
        out_shape=[jax.ShapeDtypeStruct((cout, 1), jnp.float32)] * 2,
        grid=(1,),
        in_specs=[
            pl.BlockSpec((n, cin, cin), lambda i: (0, 0, 0)),
            pl.BlockSpec((n, 1, cin), lambda i: (0, 0, 0)),
            pl.BlockSpec((cin, cout), lambda i: (0, 0)),
            pl.BlockSpec((1, cout), lambda i: (0, 0)),
            pl.BlockSpec((1, cout), lambda i: (0, 0)),
        ],
        out_specs=[pl.BlockSpec((cout, 1), lambda i: (0, 0))] * 2,
        compiler_params=pltpu.CompilerParams(
            dimension_semantics=("arbitrary",),
            vmem_limit_bytes=_VMEM_LIMIT,
        ),
    )(gmat, ssum, w2t, gamma.reshape(1, cout).astype(jnp.float32),
      beta.reshape(1, cout).astype(jnp.float32))

    ap_kern = functools.partial(_apply_kernel, hw=hw)
    out3 = pl.pallas_call(
        ap_kern,
        out_shape=jax.ShapeDtypeStruct((n, cout, hw), x.dtype),
        grid=(n // bn,),
        in_specs=[
            pl.BlockSpec((cout, cin), lambda nn: (0, 0)),
            pl.BlockSpec((cout, 1), lambda nn: (0, 0)),
            pl.BlockSpec((cout, 1), lambda nn: (0, 0)),
            pl.BlockSpec((bn, ho, wo, cin), lambda nn: (nn, 0, 0, 0)),
        ],
        out_specs=pl.BlockSpec((bn, cout, hw), lambda nn: (nn, 0, 0)),
        compiler_params=pltpu.CompilerParams(
            dimension_semantics=("parallel",),
            vmem_limit_bytes=_VMEM_LIMIT,
        ),
    )(w2, scale, shift, z)
    return out3.reshape(n, cout, ho, wo)


# speedup vs baseline: 1.5055x; 1.0223x over previous
import functools

import jax
import jax.numpy as jnp
from jax.experimental import pallas as pl
from jax.experimental.pallas import tpu as pltpu

_EPS = 1e-5
_VMEM_LIMIT = 64 * 1024 * 1024
_HIGH = jax.lax.Precision.HIGHEST


def _dw_gram_kernel(x_ref, w_ref, b_ref, z_ref, g_ref, s_ref, pad_ref,
                    *, ho, wo, ksize):
    c = x_ref.shape[-1]
    for i in range(x_ref.shape[0]):
        xin = x_ref[i]
        for wp in range(2):
            for hp in range(2):
                r0, c0 = 1 - hp, 1 - wp
                pad_ref[wp, hp, r0:r0 + ho, c0:c0 + wo, :] = xin[1 - wp, 1 - hp]
                pad_ref[wp, hp, hp * ho:hp * ho + 1, :, :] = jnp.zeros(
                    (1, wo + 1, c), jnp.float32)
                pad_ref[wp, hp, :, wp * wo:wp * wo + 1, :] = jnp.zeros(
                    (wo + 1, 1, c), jnp.float32)
        acc = jnp.zeros((ho, wo, c), jnp.float32) + b_ref[...]
        for kh in range(ksize):
            ph, a0 = kh % 2, kh // 2
            for kw in range(ksize):
                pw, b0 = kw % 2, kw // 2
                xs = pad_ref[pw, ph, a0:a0 + ho, b0:b0 + wo, :]
                acc = acc + w_ref[kh * ksize + kw] * xs
        z_ref[i] = acc
        zv = acc.reshape(ho * wo, acc.shape[-1])
        g_ref[i] = jax.lax.dot_general(zv, zv, (((0,), (0,)), ((), ())),
                                       preferred_element_type=jnp.float32,
                                       precision=_HIGH)
        s_ref[i] = jnp.sum(zv, axis=0, keepdims=True)


def _finalize_kernel(g_ref, s_ref, w2t_ref, gamma_ref, beta_ref,
                     scale_ref, shift_ref, *, count):
    g = jnp.sum(g_ref[...], axis=0)
    zs = jnp.sum(s_ref[...], axis=0)
    w2t = w2t_ref[...]
    syh = jax.lax.dot_general(zs, w2t, (((1,), (0,)), ((), ())),
                              preferred_element_type=jnp.float32,
                              precision=_HIGH)
    a = jax.lax.dot_general(g, w2t, (((1,), (0,)), ((), ())),
                            preferred_element_type=jnp.float32,
                            precision=_HIGH)
    ssq = jnp.sum(a * w2t, axis=0, keepdims=True)
    inv_cnt = jnp.float32(1.0 / count)
    mean_hat = syh * inv_cnt
    var = jnp.maximum(ssq * inv_cnt - mean_hat * mean_hat, 0.0)
    scale = gamma_ref[...] * jax.lax.rsqrt(var + _EPS)
    shift = beta_ref[...] - mean_hat * scale
    scale_ref[...] = scale.reshape(scale_ref.shape)
    shift_ref[...] = shift.reshape(shift_ref.shape)


def _apply_kernel(w2_ref, scale_ref, shift_ref, z_ref, o_ref, *, hw):
    for i in range(z_ref.shape[0]):
        zv = z_ref[i].reshape(hw, z_ref.shape[-1])
        y = jax.lax.dot_general(w2_ref[...], zv, (((1,), (1,)), ((), ())),
                                preferred_element_type=jnp.float32,
                                precision=_HIGH)
        y = y * scale_ref[...] + shift_ref[...]
        o_ref[i] = jnp.maximum(y, 0.0).astype(o_ref.dtype)


def kernel(x, dw_w, dw_b, pw_w, pw_b, gamma, beta):
    del pw_b
    n, cin, h, w = x.shape
    cout = pw_w.shape[0]
    ksize = dw_w.shape[-1]
    pad = 1
    ho = (h + 2 * pad - ksize) // 2 + 1
    wo = (w + 2 * pad - ksize) // 2 + 1
    hw = ho * wo
    bn = 4 if n % 4 == 0 else (2 if n % 2 == 0 else 1)

    xq = x.reshape(n, cin, ho, 2, wo, 2).transpose(0, 5, 3, 2, 4, 1)
    wt = dw_w.reshape(cin, ksize * ksize).T.reshape(ksize * ksize, 1, cin)
    wt = wt.astype(jnp.float32)
    bt = dw_b.reshape(1, 1, cin).astype(jnp.float32)
    w2 = pw_w.reshape(cout, cin).astype(jnp.float32)
    w2t = w2.T

    dw_kern = functools.partial(_dw_gram_kernel, ho=ho, wo=wo, ksize=ksize)
    z, gmat, ssum = pl.pallas_call(
        dw_kern,
        out_shape=[
            jax.ShapeDtypeStruct((n, ho, wo, cin), jnp.float32),
            jax.ShapeDtypeStruct((n, cin, cin), jnp.float32),
            jax.ShapeDtypeStruct((n, 1, cin), jnp.float32),
        ],
        grid=(n // bn,),
        in_specs=[
            pl.BlockSpec((bn, 2, 2, ho, wo, cin),
                         lambda nn: (nn, 0, 0, 0, 0, 0)),
            pl.BlockSpec((ksize * ksize, 1, cin), lambda nn: (0, 0, 0)),
            pl.BlockSpec((1, 1, cin), lambda nn: (0, 0, 0)),
        ],
        scratch_shapes=[
            pltpu.VMEM((2, 2, ho + 1, wo + 1, cin), jnp.float32),
        ],
        out_specs=[
            pl.BlockSpec((bn, ho, wo, cin), lambda nn: (nn, 0, 0, 0)),
            pl.BlockSpec((bn, cin, cin), lambda nn: (nn, 0, 0)),
            pl.BlockSpec((bn, 1, cin), lambda nn: (nn, 0, 0)),
        ],
        compiler_params=pltpu.CompilerParams(
            dimension_semantics=("parallel",),
            vmem_limit_bytes=_VMEM_LIMIT,
        ),
    )(xq, wt, bt)

    fin_kern = functools.partial(_finalize_kernel, count=n * hw)
    scale, shift = pl.pallas_call(
        fin_kern,
        out_shape=[jax.ShapeDtypeStruct((cout, 1), jnp.float32)] * 2,
        grid=(1,),
        in_specs=[
            pl.BlockSpec((n, cin, cin), lambda i: (0, 0, 0)),
            pl.BlockSpec((n, 1, cin), lambda i: (0, 0, 0)),
            pl.BlockSpec((cin, cout), lambda i: (0, 0)),
            pl.BlockSpec((1, cout), lambda i: (0, 0)),
            pl.BlockSpec((1, cout), lambda i: (0, 0)),
        ],
        out_specs=[pl.BlockSpec((cout, 1), lambda i: (0, 0))] * 2,
        compiler_params=pltpu.CompilerParams(
            dimension_semantics=("arbitrary",),
            vmem_limit_bytes=_VMEM_LIMIT,
        ),
    )(gmat, ssum, w2t, gamma.reshape(1, cout).astype(jnp.float32),
      beta.reshape(1, cout).astype(jnp.float32))

    ap_kern = functools.partial(_apply_kernel, hw=hw)
    out3 = pl.pallas_call(
        ap_kern,
        out_shape=jax.ShapeDtypeStruct((n, cout, hw), x.dtype),
        grid=(n // bn,),
        in_specs=[
            pl.BlockSpec((cout, cin), lambda nn: (0, 0)),
            pl.BlockSpec((cout, 1), lambda nn: (0, 0)),
            pl.BlockSpec((cout, 1), lambda nn: (0, 0)),
            pl.BlockSpec((bn, ho, wo, cin), lambda nn: (nn, 0, 0, 0)),
        ],
        out_specs=pl.BlockSpec((bn, cout, hw), lambda nn: (nn, 0, 0)),
        compiler_params=pltpu.CompilerParams(
            dimension_semantics=("parallel",),
            vmem_limit_bytes=_VMEM_LIMIT,
        ),
    )(w2, scale, shift, z)
    return out3.reshape(n, cout, ho, wo)
